# baseline (device time: 87626 ns/iter reference)
import jax
import jax.numpy as jnp
from jax import lax
from jax.experimental import pallas as pl
from jax.experimental.pallas import tpu as pltpu

N_DEV = 4
M_BLK = 2048 // N_DEV


def kernel(x, w_mat):
    m, k_per = x.shape
    _, n = w_mat.shape

    def body(x_ref, w_ref, out_ref, comm_ref, send_sems, recv_sems):
        p = lax.axis_index("i")
        left = lax.rem(p + N_DEV - 1, N_DEV)
        right = lax.rem(p + 1, N_DEV)

        barrier_sem = pltpu.get_barrier_semaphore()
        for nbr in [left, right]:
            pl.semaphore_signal(
                barrier_sem, inc=1,
                device_id=(nbr,), device_id_type=pl.DeviceIdType.MESH,
            )
        pl.semaphore_wait(barrier_sem, 2)

        w_bf = w_ref[:, :].astype(jnp.bfloat16)

        def partial(c):
            xc = x_ref[pl.ds(c * M_BLK, M_BLK), :].astype(jnp.bfloat16)
            return jnp.dot(xc, w_bf, preferred_element_type=jnp.float32)

        c0 = lax.rem(p + N_DEV - 1, N_DEV)
        comm_ref[0, :, :] = partial(c0).astype(jnp.bfloat16)

        for h in range(N_DEV - 1):
            rdma = pltpu.make_async_remote_copy(
                src_ref=comm_ref.at[h],
                dst_ref=comm_ref.at[h + 1],
                send_sem=send_sems.at[h],
                recv_sem=recv_sems.at[h],
                device_id=(right,),
                device_id_type=pl.DeviceIdType.MESH,
            )
            rdma.start()
            rdma.wait()

            c = lax.rem(p + 2 * N_DEV - h - 2, N_DEV)
            acc = comm_ref[h + 1, :, :].astype(jnp.float32) + partial(c)
            if h < N_DEV - 2:
                comm_ref[h + 1, :, :] = acc.astype(jnp.bfloat16)
            else:
                out_ref[:, :] = acc

    return pl.pallas_call(
        body,
        out_shape=jax.ShapeDtypeStruct((M_BLK, n), jnp.float32),
        in_specs=[
            pl.BlockSpec(memory_space=pltpu.VMEM),
            pl.BlockSpec(memory_space=pltpu.VMEM),
        ],
        out_specs=pl.BlockSpec(memory_space=pltpu.VMEM),
        scratch_shapes=[
            pltpu.VMEM((N_DEV, M_BLK, n), jnp.bfloat16),
            pltpu.SemaphoreType.DMA((N_DEV - 1,)),
            pltpu.SemaphoreType.DMA((N_DEV - 1,)),
        ],
        compiler_params=pltpu.CompilerParams(collective_id=0),
    )(x, w_mat)


# device time: 51653 ns/iter; 1.6964x vs baseline; 1.6964x over previous
import jax
import jax.numpy as jnp
from jax import lax
from jax.experimental import pallas as pl
from jax.experimental.pallas import tpu as pltpu

N_DEV = 4
M_BLK = 2048 // N_DEV


def kernel(x, w_mat):
    m, k_per = x.shape
    _, n = w_mat.shape
    nh = n // 2

    def body(x_ref, w_ref, out_ref,
             commR_ref, commL_ref,
             sendR_sems, recvR_sems, sendL_sems, recvL_sems):
        p = lax.axis_index("i")
        left = lax.rem(p + N_DEV - 1, N_DEV)
        right = lax.rem(p + 1, N_DEV)

        barrier_sem = pltpu.get_barrier_semaphore()
        for nbr in [left, right]:
            pl.semaphore_signal(
                barrier_sem, inc=1,
                device_id=(nbr,), device_id_type=pl.DeviceIdType.MESH,
            )
        pl.semaphore_wait(barrier_sem, 2)

        def phalf(c, half):
            xc = x_ref[pl.ds(c * M_BLK, M_BLK), :].astype(jnp.bfloat16)
            wc = w_ref[:, half * nh:(half + 1) * nh].astype(jnp.bfloat16)
            return jnp.dot(xc, wc, preferred_element_type=jnp.float32)

        def blkR(h):
            return lax.rem(p + 2 * N_DEV - h - 2, N_DEV)

        def blkL(h):
            return lax.rem(p + h + 2, N_DEV)

        commR_ref[0, :, :] = phalf(lax.rem(p + N_DEV - 1, N_DEV), 0).astype(jnp.bfloat16)
        commL_ref[0, :, :] = phalf(lax.rem(p + 1, N_DEV), 1).astype(jnp.bfloat16)

        def make_rdmas(h):
            rdmaR = pltpu.make_async_remote_copy(
                src_ref=commR_ref.at[h],
                dst_ref=commR_ref.at[h + 1],
                send_sem=sendR_sems.at[h],
                recv_sem=recvR_sems.at[h],
                device_id=(right,),
                device_id_type=pl.DeviceIdType.MESH,
            )
            rdmaL = pltpu.make_async_remote_copy(
                src_ref=commL_ref.at[h],
                dst_ref=commL_ref.at[h + 1],
                send_sem=sendL_sems.at[h],
                recv_sem=recvL_sems.at[h],
                device_id=(left,),
                device_id_type=pl.DeviceIdType.MESH,
            )
            return rdmaR, rdmaL

        for h in range(N_DEV - 1):
            rdmaR, rdmaL = make_rdmas(h)
            rdmaR.start()
            rdmaL.start()
            pR = phalf(blkR(h), 0)
            pL = phalf(blkL(h), 1)
            rdmaR.wait()
            rdmaL.wait()
            accR = commR_ref[h + 1, :, :].astype(jnp.float32) + pR
            accL = commL_ref[h + 1, :, :].astype(jnp.float32) + pL
            if h < N_DEV - 2:
                commR_ref[h + 1, :, :] = accR.astype(jnp.bfloat16)
                commL_ref[h + 1, :, :] = accL.astype(jnp.bfloat16)
            else:
                out_ref[:, 0:nh] = accR
                out_ref[:, nh:n] = accL

    return pl.pallas_call(
        body,
        out_shape=jax.ShapeDtypeStruct((M_BLK, n), jnp.float32),
        in_specs=[
            pl.BlockSpec(memory_space=pltpu.VMEM),
            pl.BlockSpec(memory_space=pltpu.VMEM),
        ],
        out_specs=pl.BlockSpec(memory_space=pltpu.VMEM),
        scratch_shapes=[
            pltpu.VMEM((N_DEV, M_BLK, nh), jnp.bfloat16),
            pltpu.VMEM((N_DEV, M_BLK, nh), jnp.bfloat16),
            pltpu.SemaphoreType.DMA((N_DEV - 1,)),
            pltpu.SemaphoreType.DMA((N_DEV - 1,)),
            pltpu.SemaphoreType.DMA((N_DEV - 1,)),
            pltpu.SemaphoreType.DMA((N_DEV - 1,)),
        ],
        compiler_params=pltpu.CompilerParams(collective_id=0),
    )(x, w_mat)


# device time: 46313 ns/iter; 1.8920x vs baseline; 1.1153x over previous
import jax
import jax.numpy as jnp
from jax import lax
from jax.experimental import pallas as pl
from jax.experimental.pallas import tpu as pltpu

N_DEV = 4
M_BLK = 2048 // N_DEV
S = 2


def kernel(x, w_mat):
    m, k_per = x.shape
    _, n = w_mat.shape
    nh = n // 2
    swid = nh // S

    def body(x_ref, w_ref, out_ref,
             commR_ref, commL_ref,
             sendR_sems, recvR_sems, sendL_sems, recvL_sems):
        p = lax.axis_index("i")
        left = lax.rem(p + N_DEV - 1, N_DEV)
        right = lax.rem(p + 1, N_DEV)

        barrier_sem = pltpu.get_barrier_semaphore()
        for nbr in [left, right]:
            pl.semaphore_signal(
                barrier_sem, inc=1,
                device_id=(nbr,), device_id_type=pl.DeviceIdType.MESH,
            )
        pl.semaphore_wait(barrier_sem, 2)

        def pq(c, q):
            xc = x_ref[pl.ds(c * M_BLK, M_BLK), :].astype(jnp.bfloat16)
            wc = w_ref[:, q * swid:(q + 1) * swid].astype(jnp.bfloat16)
            return jnp.dot(xc, wc, preferred_element_type=jnp.float32)

        def blkR(h):
            return lax.rem(p + 2 * N_DEV - h - 2, N_DEV)

        def blkL(h):
            return lax.rem(p + h + 2, N_DEV)

        def mk(comm_ref, sems_s, sems_r, h, j, dst):
            return pltpu.make_async_remote_copy(
                src_ref=comm_ref.at[h, j],
                dst_ref=comm_ref.at[h + 1, j],
                send_sem=sems_s.at[h, j],
                recv_sem=sems_r.at[h, j],
                device_id=(dst,),
                device_id_type=pl.DeviceIdType.MESH,
            )

        all_rdmas = []

        bR0 = lax.rem(p + N_DEV - 1, N_DEV)
        bL0 = lax.rem(p + 1, N_DEV)
        curR, curL = [], []
        for j in range(S):
            commR_ref[0, j] = pq(bR0, j).astype(jnp.bfloat16)
            dR = mk(commR_ref, sendR_sems, recvR_sems, 0, j, right)
            dR.start()
            commL_ref[0, j] = pq(bL0, S + j).astype(jnp.bfloat16)
            dL = mk(commL_ref, sendL_sems, recvL_sems, 0, j, left)
            dL.start()
            curR.append(dR)
            curL.append(dL)
            all_rdmas.extend([dR, dL])

        pR = [pq(blkR(0), j) for j in range(S)]
        pL = [pq(blkL(0), S + j) for j in range(S)]

        for h in range(N_DEV - 1):
            nextR, nextL = [], []
            for j in range(S):
                curR[j].wait_recv()
                accR = commR_ref[h + 1, j].astype(jnp.float32) + pR[j]
                if h < N_DEV - 2:
                    commR_ref[h + 1, j] = accR.astype(jnp.bfloat16)
                    dR = mk(commR_ref, sendR_sems, recvR_sems, h + 1, j, right)
                    dR.start()
                    nextR.append(dR)
                    all_rdmas.append(dR)
                else:
                    out_ref[:, j * swid:(j + 1) * swid] = accR

                curL[j].wait_recv()
                accL = commL_ref[h + 1, j].astype(jnp.float32) + pL[j]
                if h < N_DEV - 2:
                    commL_ref[h + 1, j] = accL.astype(jnp.bfloat16)
                    dL = mk(commL_ref, sendL_sems, recvL_sems, h + 1, j, left)
                    dL.start()
                    nextL.append(dL)
                    all_rdmas.append(dL)
                else:
                    out_ref[:, nh + j * swid:nh + (j + 1) * swid] = accL
            curR, curL = nextR, nextL
            if h < N_DEV - 2:
                pR = [pq(blkR(h + 1), j) for j in range(S)]
                pL = [pq(blkL(h + 1), S + j) for j in range(S)]

        for d in all_rdmas:
            d.wait_send()

    return pl.pallas_call(
        body,
        out_shape=jax.ShapeDtypeStruct((M_BLK, n), jnp.float32),
        in_specs=[
            pl.BlockSpec(memory_space=pltpu.VMEM),
            pl.BlockSpec(memory_space=pltpu.VMEM),
        ],
        out_specs=pl.BlockSpec(memory_space=pltpu.VMEM),
        scratch_shapes=[
            pltpu.VMEM((N_DEV, S, M_BLK, swid), jnp.bfloat16),
            pltpu.VMEM((N_DEV, S, M_BLK, swid), jnp.bfloat16),
            pltpu.SemaphoreType.DMA((N_DEV - 1, S)),
            pltpu.SemaphoreType.DMA((N_DEV - 1, S)),
            pltpu.SemaphoreType.DMA((N_DEV - 1, S)),
            pltpu.SemaphoreType.DMA((N_DEV - 1, S)),
        ],
        compiler_params=pltpu.CompilerParams(collective_id=0),
    )(x, w_mat)
